# baseline (speedup 1.0000x reference)
import functools

import jax
import jax.numpy as jnp
from jax import lax
from jax.experimental import pallas as pl
from jax.experimental.pallas import tpu as pltpu
from jax.experimental.pallas import tpu_sc as plsc

N_NODES = 100000
D = 128
B = 500000

_info = plsc.get_sparse_core_info()
NC = _info.num_cores
NS = _info.num_subcores
NW = NC * NS
C = 80
NCHUNKS = B // C
CHUNKS_PER_W = -(-NCHUNKS // NW)
NBUF = 6
AHEAD = 3
SLOTS = 2


def _sum_body(a_ref, b_ref, o_ref):
    o_ref[...] = a_ref[...] + b_ref[...]


def _sum_table(node_features, memory):
    rows = 10000
    return pl.pallas_call(
        _sum_body,
        grid=(N_NODES // rows,),
        in_specs=[pl.BlockSpec((rows, D), lambda i: (i, 0)),
                  pl.BlockSpec((rows, D), lambda i: (i, 0))],
        out_specs=pl.BlockSpec((rows, D), lambda i: (i, 0)),
        out_shape=jax.ShapeDtypeStruct((N_NODES, D), jnp.float32),
    )(node_features, memory)


_mesh = plsc.VectorSubcoreMesh(core_axis_name="c", subcore_axis_name="s")


@functools.partial(
    pl.kernel,
    mesh=_mesh,
    out_type=jax.ShapeDtypeStruct((B, D), jnp.float32),
    scratch_types=(
        [pltpu.VMEM((C,), jnp.int32)] * NBUF
        + [pltpu.VMEM((C, D), jnp.float32)] * NBUF
        + [pltpu.VMEM_SHARED((NS, SLOTS, C, D), jnp.float32)]
        + [pltpu.SemaphoreType.DMA] * NBUF
        + [pltpu.SemaphoreType.DMA] * NBUF
        + [pltpu.SemaphoreType.DMA] * SLOTS
        + [pltpu.SemaphoreType.DMA] * SLOTS
    ),
)
def _gather_k(table_hbm, idx_hbm, out_hbm, *scratch):
    idxs = scratch[:NBUF]
    bufs = scratch[NBUF:2 * NBUF]
    sp = scratch[2 * NBUF]
    isem = scratch[2 * NBUF + 1:3 * NBUF + 1]
    gsem = scratch[3 * NBUF + 1:4 * NBUF + 1]
    psem = scratch[4 * NBUF + 1:4 * NBUF + 1 + SLOTS]
    fsem = scratch[4 * NBUF + 1 + SLOTS:]
    sid = lax.axis_index("s")
    wid = sid * NC + lax.axis_index("c")

    def active(k):
        return jnp.logical_and(k >= 0,
                               jnp.logical_and(k < CHUNKS_PER_W,
                                               wid + k * NW < NCHUNKS))

    pltpu.async_copy(idx_hbm.at[pl.ds(wid * C, C)], idxs[0], isem[0])

    def step(t, u):
        up = (u + 1) % NBUF

        @pl.when(active(t + 1))
        def _():
            pltpu.async_copy(
                idx_hbm.at[pl.ds((wid + (t + 1) * NW) * C, C)],
                idxs[up], isem[up])

        @pl.when(active(t))
        def _():
            pltpu.make_async_copy(idx_hbm.at[pl.ds((wid + t * NW) * C, C)],
                                  idxs[u], isem[u]).wait()
            pltpu.async_copy(table_hbm.at[idxs[u]], bufs[u], gsem[u])

        tf = t - AHEAD - SLOTS
        sf = (u - AHEAD) % SLOTS

        @pl.when(active(tf))
        def _():
            fcid = wid + tf * NW
            pltpu.make_async_copy(sp.at[sid, sf],
                                  out_hbm.at[pl.ds(fcid * C, C)],
                                  fsem[sf]).wait()

        tp = t - AHEAD
        ub = (u - AHEAD) % NBUF

        @pl.when(active(tp))
        def _():
            pltpu.make_async_copy(table_hbm.at[idxs[ub]], bufs[ub],
                                  gsem[ub]).wait()
            pltpu.async_copy(bufs[ub], sp.at[sid, sf], psem[sf])

        td = t - AHEAD - 1
        ud = (u - AHEAD - 1) % NBUF
        sd = (u - AHEAD - 1) % SLOTS

        @pl.when(active(td))
        def _():
            pltpu.make_async_copy(bufs[ud], sp.at[sid, sd],
                                  psem[sd]).wait()
            dcid = wid + td * NW
            pltpu.async_copy(sp.at[sid, sd],
                             out_hbm.at[pl.ds(dcid * C, C)], fsem[sd])

    NSTEP = CHUNKS_PER_W + AHEAD + SLOTS + 1
    NITER = -(-NSTEP // NBUF)

    def outer(j, _):
        for u in range(NBUF):
            step(j * NBUF + u, u)
        return ()

    lax.fori_loop(0, NITER, outer, ())


def kernel(node_features, memory, source_nodes, timestamps, time_w, time_b):
    table = _sum_table(node_features, memory)
    idx = source_nodes.astype(jnp.int32)
    return _gather_k(table, idx)
